# baseline (speedup 1.0000x reference)
import functools

import jax
import jax.numpy as jnp
from jax import lax
from jax.experimental import pallas as pl
from jax.experimental.pallas import tpu as pltpu, tpu_sc as plsc

N_NODES = 10000
N_EDGES = 320000
D = 128
BN_EPS = 1e-5

NPAD = 10240
DUMMY = 10000
NC, NS = 2, 16
NW = NC * NS
CH = 128
EPT = 10240
C_CHUNKS = EPT // CH
PH = 2
PC = C_CHUNKS // PH
E_PAD = NW * EPT
STRIPE = NPAD // NS

def _sc_degree_body(dst_hbm, zeros1_hbm, out_hbm, idx_v, ones_v, deg_sh):
    c = lax.axis_index("c")
    s = lax.axis_index("s")
    wid = c * NS + s
    pltpu.sync_copy(zeros1_hbm.at[pl.ds(s * STRIPE, STRIPE)],
                    deg_sh.at[pl.ds(s * STRIPE, STRIPE)])
    pltpu.sync_copy(dst_hbm.at[wid], idx_v)
    for k in range(CH // 16):
        ones_v[pl.ds(k * 16, 16)] = jnp.ones((16,), jnp.float32)
    plsc.subcore_barrier()

    def chunk(j, carry):
        pltpu.sync_copy(ones_v, deg_sh.at[idx_v.at[j]], add=True)
        return carry

    lax.fori_loop(0, C_CHUNKS, chunk, 0)
    plsc.subcore_barrier()
    pltpu.sync_copy(deg_sh.at[pl.ds(s * STRIPE, STRIPE)],
                    out_hbm.at[c, pl.ds(s * STRIPE, STRIPE)])


NBUF = 2


def _sc_scatter_body(hs_hbm, src_hbm, dst_hbm, zeros2_hbm, out_hbm,
                     si_v, di_v, rows_v, acc_sh, gsem, ssem):
    c = lax.axis_index("c")
    s = lax.axis_index("s")
    wid = c * NS + s
    pltpu.sync_copy(zeros2_hbm.at[pl.ds(s * STRIPE, STRIPE)],
                    acc_sh.at[pl.ds(s * STRIPE, STRIPE)])
    plsc.subcore_barrier()

    for p in range(PH):
        pltpu.sync_copy(src_hbm.at[wid, pl.ds(p * PC, PC)], si_v)
        pltpu.sync_copy(dst_hbm.at[wid, pl.ds(p * PC, PC)], di_v)
        for b in range(NBUF):
            pltpu.async_copy(hs_hbm.at[si_v.at[b]], rows_v.at[b], gsem.at[b])

        def group(g, carry):
            j0 = g * NBUF
            for b in range(NBUF):
                j = j0 + b
                pltpu.make_async_copy(hs_hbm.at[si_v.at[j]], rows_v.at[b],
                                      gsem.at[b]).wait()
                pltpu.async_copy(rows_v.at[b], acc_sh.at[di_v.at[j]],
                                 ssem.at[b], add=True)
            for b in range(NBUF):
                j = j0 + b
                jn = j + NBUF

                @pl.when(jn < PC)
                def _():
                    pltpu.make_async_copy(rows_v.at[b],
                                          acc_sh.at[di_v.at[j]],
                                          ssem.at[b]).wait()
                    pltpu.async_copy(hs_hbm.at[si_v.at[jn]], rows_v.at[b],
                                     gsem.at[b])
            return carry

        lax.fori_loop(0, PC // NBUF, group, 0)
        for b in range(NBUF):
            pltpu.make_async_copy(rows_v.at[b],
                                  acc_sh.at[di_v.at[PC - NBUF + b]],
                                  ssem.at[b]).wait()
    plsc.subcore_barrier()
    pltpu.sync_copy(acc_sh.at[pl.ds(s * STRIPE, STRIPE)],
                    out_hbm.at[c, pl.ds(s * STRIPE, STRIPE)])


@functools.lru_cache(maxsize=1)
def _sc_kernels():
    mesh = plsc.VectorSubcoreMesh(core_axis_name="c", subcore_axis_name="s",
                                  num_cores=NC, num_subcores=NS)
    sc_degree = pl.kernel(
        _sc_degree_body,
        out_type=jax.ShapeDtypeStruct((NC, NPAD), jnp.float32),
        mesh=mesh,
        scratch_types=[
            pltpu.VMEM((C_CHUNKS, CH), jnp.int32),
            pltpu.VMEM((CH,), jnp.float32),
            pltpu.VMEM_SHARED((NPAD,), jnp.float32),
        ],
    )
    sc_scatter = pl.kernel(
        _sc_scatter_body,
        out_type=jax.ShapeDtypeStruct((NC, NPAD, D), jnp.float32),
        mesh=mesh,
        scratch_types=[
            pltpu.VMEM((PC, CH), jnp.int32),
            pltpu.VMEM((PC, CH), jnp.int32),
            pltpu.VMEM((NBUF, CH, D), jnp.float32),
            pltpu.VMEM_SHARED((NPAD, D), jnp.float32),
            pltpu.SemaphoreType.DMA((NBUF,)),
            pltpu.SemaphoreType.DMA((NBUF,)),
        ],
    )
    return sc_degree, sc_scatter


ROWS = 1024
GRID = NPAD // ROWS

_row = pl.BlockSpec((ROWS, D), lambda i: (i, 0))
_wmat = pl.BlockSpec((D, D), lambda i: (0, 0))
_vec = pl.BlockSpec((1, D), lambda i: (0, 0))
_deg = pl.BlockSpec((NC, ROWS, 1), lambda i: (0, i, 0))
_acc = pl.BlockSpec((NC, ROWS, D), lambda i: (0, i, 0))


def _dinv(deg_ref):
    return lax.rsqrt(deg_ref[0] + deg_ref[1] + 1.0)


def _elu(v):
    return jnp.where(v > 0, v, jnp.exp(jnp.minimum(v, 0.0)) - 1.0)


def _tc_pre_body(x_ref, w1_ref, ws_ref, bs_ref, deg_ref, hs_ref, xi_ref):
    xb = x_ref[...]
    di = _dinv(deg_ref)
    h = jnp.dot(xb, w1_ref[...], preferred_element_type=jnp.float32)
    hs_ref[...] = h * di
    xi_ref[...] = jnp.dot(xb, ws_ref[...],
                          preferred_element_type=jnp.float32) + bs_ref[...]


_tc_pre = pl.pallas_call(
    _tc_pre_body,
    grid=(GRID,),
    in_specs=[_row, _wmat, _wmat, _vec, _deg],
    out_specs=[_row, _row],
    out_shape=[jax.ShapeDtypeStruct((NPAD, D), jnp.float32),
               jax.ShapeDtypeStruct((NPAD, D), jnp.float32)],
)


def _combine(acc_ref, hs_ref, deg_ref, b_ref, g_ref, be_ref):
    di = _dinv(deg_ref)
    conv = di * (acc_ref[0] + acc_ref[1] + hs_ref[...]) + b_ref[...]
    gs = g_ref[...] * lax.rsqrt(jnp.float32(1.0 + BN_EPS))
    return _elu(conv * gs + be_ref[...])


def _tc_mid_body(acc_ref, hs_ref, deg_ref, b_ref, g_ref, be_ref, skip_ref,
                 w_ref, ws_ref, bs_ref, hsn_ref, sk_ref):
    h = _combine(acc_ref, hs_ref, deg_ref, b_ref, g_ref, be_ref) + skip_ref[...]
    di = _dinv(deg_ref)
    hsn_ref[...] = di * jnp.dot(h, w_ref[...],
                                preferred_element_type=jnp.float32)
    sk_ref[...] = jnp.dot(h, ws_ref[...],
                          preferred_element_type=jnp.float32) + bs_ref[...]


_tc_mid = pl.pallas_call(
    _tc_mid_body,
    grid=(GRID,),
    in_specs=[_acc, _row, _deg, _vec, _vec, _vec, _row, _wmat, _wmat, _vec],
    out_specs=[_row, _row],
    out_shape=[jax.ShapeDtypeStruct((NPAD, D), jnp.float32),
               jax.ShapeDtypeStruct((NPAD, D), jnp.float32)],
)


def _tc_mid2_body(acc_ref, hs_ref, deg_ref, b_ref, g_ref, be_ref, skip_ref,
                  w_ref, hsn_ref):
    h = _combine(acc_ref, hs_ref, deg_ref, b_ref, g_ref, be_ref) + skip_ref[...]
    di = _dinv(deg_ref)
    hsn_ref[...] = di * jnp.dot(h, w_ref[...],
                                preferred_element_type=jnp.float32)


_tc_mid2 = pl.pallas_call(
    _tc_mid2_body,
    grid=(GRID,),
    in_specs=[_acc, _row, _deg, _vec, _vec, _vec, _row, _wmat],
    out_specs=_row,
    out_shape=jax.ShapeDtypeStruct((NPAD, D), jnp.float32),
)


def _tc_post_body(acc_ref, hs_ref, deg_ref, b_ref, g_ref, be_ref, out_ref):
    out_ref[...] = _combine(acc_ref, hs_ref, deg_ref, b_ref, g_ref, be_ref)


_tc_post = pl.pallas_call(
    _tc_post_body,
    grid=(GRID,),
    in_specs=[_acc, _row, _deg, _vec, _vec, _vec],
    out_specs=_row,
    out_shape=jax.ShapeDtypeStruct((NPAD, D), jnp.float32),
)


def kernel(x, edge_index, W1, b1, W2, b2, W3, b3, g1, be1, g2, be2, g3, be3,
           Ws1, bs1, Ws2, bs2):
    src = edge_index[0].astype(jnp.int32)
    dst = edge_index[1].astype(jnp.int32)
    pad = DUMMY + jnp.arange(E_PAD - N_EDGES, dtype=jnp.int32) % (NPAD - DUMMY)
    src_t = jnp.concatenate([src, pad]).reshape(NW, C_CHUNKS, CH)
    dst_t = jnp.concatenate([dst, pad]).reshape(NW, C_CHUNKS, CH)
    x_p = jnp.pad(x, ((0, NPAD - N_NODES), (0, 0)))
    zeros1 = jnp.zeros((NPAD,), jnp.float32)
    zeros2 = jnp.zeros((NPAD, D), jnp.float32)
    row = lambda v: v.reshape(1, D)
    _sc_degree, _sc_scatter = _sc_kernels()

    deg = _sc_degree(dst_t, zeros1).reshape(NC, NPAD, 1)

    hs1, x_init = _tc_pre(x_p, W1, Ws1, row(bs1), deg)
    acc1 = _sc_scatter(hs1, src_t, dst_t, zeros2)
    hs2, x_skip = _tc_mid(acc1, hs1, deg, row(b1), row(g1), row(be1), x_init,
                          W2, Ws2, row(bs2))
    acc2 = _sc_scatter(hs2, src_t, dst_t, zeros2)
    hs3 = _tc_mid2(acc2, hs2, deg, row(b2), row(g2), row(be2), x_skip, W3)
    acc3 = _sc_scatter(hs3, src_t, dst_t, zeros2)
    h3 = _tc_post(acc3, hs3, deg, row(b3), row(g3), row(be3))
    return h3[:N_NODES]


# speedup vs baseline: 1.2561x; 1.2561x over previous
import functools

import jax
import jax.numpy as jnp
from jax import lax
from jax.experimental import pallas as pl
from jax.experimental.pallas import tpu as pltpu, tpu_sc as plsc

N_NODES = 10000
N_EDGES = 320000
D = 128
BN_EPS = 1e-5

NPAD = 10240
DUMMY = 10000
NC, NS = 2, 16
NW = NC * NS
CH = 128
EPT = 10240
C_CHUNKS = EPT // CH
PH = 2
PC = C_CHUNKS // PH
E_PAD = NW * EPT
STRIPE = NPAD // NS

def _sc_degree_body(dst_hbm, zeros1_hbm, out_hbm, idx_v, ones_v, deg_sh):
    c = lax.axis_index("c")
    s = lax.axis_index("s")
    wid = c * NS + s
    pltpu.sync_copy(zeros1_hbm.at[pl.ds(s * STRIPE, STRIPE)],
                    deg_sh.at[pl.ds(s * STRIPE, STRIPE)])
    pltpu.sync_copy(dst_hbm.at[wid], idx_v)
    for k in range(CH // 16):
        ones_v[pl.ds(k * 16, 16)] = jnp.ones((16,), jnp.float32)
    plsc.subcore_barrier()

    def chunk(j, carry):
        pltpu.sync_copy(ones_v, deg_sh.at[idx_v.at[j]], add=True)
        return carry

    lax.fori_loop(0, C_CHUNKS, chunk, 0)
    plsc.subcore_barrier()
    pltpu.sync_copy(deg_sh.at[pl.ds(s * STRIPE, STRIPE)],
                    out_hbm.at[c, pl.ds(s * STRIPE, STRIPE)])


NBUF = 2


def _sc_scatter_body(hs_hbm, src_hbm, dst_hbm, zeros2_hbm, out_hbm,
                     si_v, di_v, rows_v, acc_sh, gsem):
    c = lax.axis_index("c")
    s = lax.axis_index("s")
    wid = c * NS + s
    pltpu.sync_copy(zeros2_hbm.at[pl.ds(s * STRIPE, STRIPE)],
                    acc_sh.at[pl.ds(s * STRIPE, STRIPE)])
    plsc.subcore_barrier()

    for p in range(PH):
        pltpu.sync_copy(src_hbm.at[wid, pl.ds(p * PC, PC)], si_v)
        pltpu.sync_copy(dst_hbm.at[wid, pl.ds(p * PC, PC)], di_v)
        for b in range(NBUF):
            pltpu.async_copy(hs_hbm.at[si_v.at[b]], rows_v.at[b], gsem.at[b])

        def group(g, carry):
            j0 = g * NBUF
            for b in range(NBUF):
                j = j0 + b
                pltpu.make_async_copy(hs_hbm.at[si_v.at[j]], rows_v.at[b],
                                      gsem.at[b]).wait()
                pltpu.sync_copy(rows_v.at[b], acc_sh.at[di_v.at[j]],
                                add=True)
                jn = j + NBUF

                @pl.when(jn < PC)
                def _():
                    pltpu.async_copy(hs_hbm.at[si_v.at[jn]], rows_v.at[b],
                                     gsem.at[b])
            return carry

        lax.fori_loop(0, PC // NBUF, group, 0)
    plsc.subcore_barrier()
    pltpu.sync_copy(acc_sh.at[pl.ds(s * STRIPE, STRIPE)],
                    out_hbm.at[c, pl.ds(s * STRIPE, STRIPE)])


@functools.lru_cache(maxsize=1)
def _sc_kernels():
    mesh = plsc.VectorSubcoreMesh(core_axis_name="c", subcore_axis_name="s",
                                  num_cores=NC, num_subcores=NS)
    sc_degree = pl.kernel(
        _sc_degree_body,
        out_type=jax.ShapeDtypeStruct((NC, NPAD), jnp.float32),
        mesh=mesh,
        scratch_types=[
            pltpu.VMEM((C_CHUNKS, CH), jnp.int32),
            pltpu.VMEM((CH,), jnp.float32),
            pltpu.VMEM_SHARED((NPAD,), jnp.float32),
        ],
    )
    sc_scatter = pl.kernel(
        _sc_scatter_body,
        out_type=jax.ShapeDtypeStruct((NC, NPAD, D), jnp.float32),
        mesh=mesh,
        scratch_types=[
            pltpu.VMEM((PC, CH), jnp.int32),
            pltpu.VMEM((PC, CH), jnp.int32),
            pltpu.VMEM((NBUF, CH, D), jnp.float32),
            pltpu.VMEM_SHARED((NPAD, D), jnp.float32),
            pltpu.SemaphoreType.DMA((NBUF,)),
        ],
    )
    return sc_degree, sc_scatter


ROWS = 2048
GRID = NPAD // ROWS

_row = pl.BlockSpec((ROWS, D), lambda i: (i, 0))
_wmat = pl.BlockSpec((D, D), lambda i: (0, 0))
_vec = pl.BlockSpec((1, D), lambda i: (0, 0))
_deg = pl.BlockSpec((NC, ROWS, 1), lambda i: (0, i, 0))
_acc = pl.BlockSpec((NC, ROWS, D), lambda i: (0, i, 0))


def _dinv(deg_ref):
    return lax.rsqrt(deg_ref[0] + deg_ref[1] + 1.0)


def _elu(v):
    return jnp.where(v > 0, v, jnp.exp(jnp.minimum(v, 0.0)) - 1.0)


def _tc_pre_body(x_ref, w1_ref, ws_ref, bs_ref, deg_ref, hs_ref, xi_ref):
    xb = x_ref[...]
    di = _dinv(deg_ref)
    h = jnp.dot(xb, w1_ref[...], preferred_element_type=jnp.float32)
    hs_ref[...] = h * di
    xi_ref[...] = jnp.dot(xb, ws_ref[...],
                          preferred_element_type=jnp.float32) + bs_ref[...]


_tc_pre = pl.pallas_call(
    _tc_pre_body,
    grid=(GRID,),
    in_specs=[_row, _wmat, _wmat, _vec, _deg],
    out_specs=[_row, _row],
    out_shape=[jax.ShapeDtypeStruct((NPAD, D), jnp.float32),
               jax.ShapeDtypeStruct((NPAD, D), jnp.float32)],
)


def _combine(acc_ref, hs_ref, deg_ref, b_ref, g_ref, be_ref):
    di = _dinv(deg_ref)
    conv = di * (acc_ref[0] + acc_ref[1] + hs_ref[...]) + b_ref[...]
    gs = g_ref[...] * lax.rsqrt(jnp.float32(1.0 + BN_EPS))
    return _elu(conv * gs + be_ref[...])


def _tc_mid_body(acc_ref, hs_ref, deg_ref, b_ref, g_ref, be_ref, skip_ref,
                 w_ref, ws_ref, bs_ref, hsn_ref, sk_ref):
    h = _combine(acc_ref, hs_ref, deg_ref, b_ref, g_ref, be_ref) + skip_ref[...]
    di = _dinv(deg_ref)
    hsn_ref[...] = di * jnp.dot(h, w_ref[...],
                                preferred_element_type=jnp.float32)
    sk_ref[...] = jnp.dot(h, ws_ref[...],
                          preferred_element_type=jnp.float32) + bs_ref[...]


_tc_mid = pl.pallas_call(
    _tc_mid_body,
    grid=(GRID,),
    in_specs=[_acc, _row, _deg, _vec, _vec, _vec, _row, _wmat, _wmat, _vec],
    out_specs=[_row, _row],
    out_shape=[jax.ShapeDtypeStruct((NPAD, D), jnp.float32),
               jax.ShapeDtypeStruct((NPAD, D), jnp.float32)],
)


def _tc_mid2_body(acc_ref, hs_ref, deg_ref, b_ref, g_ref, be_ref, skip_ref,
                  w_ref, hsn_ref):
    h = _combine(acc_ref, hs_ref, deg_ref, b_ref, g_ref, be_ref) + skip_ref[...]
    di = _dinv(deg_ref)
    hsn_ref[...] = di * jnp.dot(h, w_ref[...],
                                preferred_element_type=jnp.float32)


_tc_mid2 = pl.pallas_call(
    _tc_mid2_body,
    grid=(GRID,),
    in_specs=[_acc, _row, _deg, _vec, _vec, _vec, _row, _wmat],
    out_specs=_row,
    out_shape=jax.ShapeDtypeStruct((NPAD, D), jnp.float32),
)


def _tc_post_body(acc_ref, hs_ref, deg_ref, b_ref, g_ref, be_ref, out_ref):
    out_ref[...] = _combine(acc_ref, hs_ref, deg_ref, b_ref, g_ref, be_ref)


_tc_post = pl.pallas_call(
    _tc_post_body,
    grid=(GRID,),
    in_specs=[_acc, _row, _deg, _vec, _vec, _vec],
    out_specs=_row,
    out_shape=jax.ShapeDtypeStruct((NPAD, D), jnp.float32),
)


def kernel(x, edge_index, W1, b1, W2, b2, W3, b3, g1, be1, g2, be2, g3, be3,
           Ws1, bs1, Ws2, bs2):
    src = edge_index[0].astype(jnp.int32)
    dst = edge_index[1].astype(jnp.int32)
    pad = DUMMY + jnp.arange(E_PAD - N_EDGES, dtype=jnp.int32) % (NPAD - DUMMY)
    src_t = jnp.concatenate([src, pad]).reshape(NW, C_CHUNKS, CH)
    dst_t = jnp.concatenate([dst, pad]).reshape(NW, C_CHUNKS, CH)
    x_p = jnp.pad(x, ((0, NPAD - N_NODES), (0, 0)))
    zeros1 = jnp.zeros((NPAD,), jnp.float32)
    zeros2 = jnp.zeros((NPAD, D), jnp.float32)
    row = lambda v: v.reshape(1, D)
    _sc_degree, _sc_scatter = _sc_kernels()

    deg = _sc_degree(dst_t, zeros1).reshape(NC, NPAD, 1)

    hs1, x_init = _tc_pre(x_p, W1, Ws1, row(bs1), deg)
    acc1 = _sc_scatter(hs1, src_t, dst_t, zeros2)
    hs2, x_skip = _tc_mid(acc1, hs1, deg, row(b1), row(g1), row(be1), x_init,
                          W2, Ws2, row(bs2))
    acc2 = _sc_scatter(hs2, src_t, dst_t, zeros2)
    hs3 = _tc_mid2(acc2, hs2, deg, row(b2), row(g2), row(be2), x_skip, W3)
    acc3 = _sc_scatter(hs3, src_t, dst_t, zeros2)
    h3 = _tc_post(acc3, hs3, deg, row(b3), row(g3), row(be3))
    return h3[:N_NODES]


# speedup vs baseline: 1.2876x; 1.0250x over previous
import functools

import jax
import jax.numpy as jnp
from jax import lax
from jax.experimental import pallas as pl
from jax.experimental.pallas import tpu as pltpu, tpu_sc as plsc

N_NODES = 10000
N_EDGES = 320000
D = 128
BN_EPS = 1e-5

NPAD = 10240
DUMMY = 10000
NC, NS = 2, 16
NW = NC * NS
CH = 128
EPT = 10240
C_CHUNKS = EPT // CH
PH = 2
PC = C_CHUNKS // PH
E_PAD = NW * EPT
STRIPE = NPAD // NS

def _sc_degree_body(dst_hbm, zeros1_hbm, out_hbm, idx_v, ones_v, deg_sh):
    c = lax.axis_index("c")
    s = lax.axis_index("s")
    wid = c * NS + s
    pltpu.sync_copy(zeros1_hbm.at[pl.ds(s * STRIPE, STRIPE)],
                    deg_sh.at[pl.ds(s * STRIPE, STRIPE)])
    pltpu.sync_copy(dst_hbm.at[wid], idx_v)
    for k in range(CH // 16):
        ones_v[pl.ds(k * 16, 16)] = jnp.ones((16,), jnp.float32)
    plsc.subcore_barrier()

    def chunk(j, carry):
        pltpu.sync_copy(ones_v, deg_sh.at[idx_v.at[j]], add=True)
        return carry

    lax.fori_loop(0, C_CHUNKS, chunk, 0)
    plsc.subcore_barrier()
    pltpu.sync_copy(deg_sh.at[pl.ds(s * STRIPE, STRIPE)],
                    out_hbm.at[c, pl.ds(s * STRIPE, STRIPE)])


NBUF = 2


def _sc_scatter_body(hs_hbm, src_hbm, dst_hbm, out_hbm,
                     si_v, di_v, rows_v, acc_sh, gsem):
    c = lax.axis_index("c")
    s = lax.axis_index("s")
    wid = c * NS + s
    zv = jnp.zeros((16,), jnp.float32)

    def zrow(i, carry):
        for k in range(D // 16):
            rows_v[i, pl.ds(k * 16, 16)] = zv
        return carry

    lax.fori_loop(0, NBUF * CH, zrow, 0)
    ring_rows = NBUF * CH
    off = 0
    while off < STRIPE:
        n = min(ring_rows, STRIPE - off)
        pltpu.sync_copy(rows_v.at[pl.ds(0, n)],
                        acc_sh.at[pl.ds(s * STRIPE + off, n)])
        off += n
    plsc.subcore_barrier()

    for p in range(PH):
        pltpu.sync_copy(src_hbm.at[wid, pl.ds(p * PC, PC)], si_v)
        pltpu.sync_copy(dst_hbm.at[wid, pl.ds(p * PC, PC)], di_v)
        for b in range(NBUF):
            pltpu.async_copy(hs_hbm.at[si_v.at[b]],
                             rows_v.at[pl.ds(b * CH, CH)], gsem.at[b])

        def group(g, carry):
            j0 = g * NBUF
            for b in range(NBUF):
                j = j0 + b
                pltpu.make_async_copy(hs_hbm.at[si_v.at[j]],
                                      rows_v.at[pl.ds(b * CH, CH)],
                                      gsem.at[b]).wait()
                pltpu.sync_copy(rows_v.at[pl.ds(b * CH, CH)],
                                acc_sh.at[di_v.at[j]], add=True)
                jn = j + NBUF

                @pl.when(jn < PC)
                def _():
                    pltpu.async_copy(hs_hbm.at[si_v.at[jn]],
                                     rows_v.at[pl.ds(b * CH, CH)],
                                     gsem.at[b])
            return carry

        lax.fori_loop(0, PC // NBUF, group, 0)
    plsc.subcore_barrier()
    pltpu.sync_copy(acc_sh.at[pl.ds(s * STRIPE, STRIPE)],
                    out_hbm.at[c, pl.ds(s * STRIPE, STRIPE)])


@functools.lru_cache(maxsize=1)
def _sc_kernels():
    mesh = plsc.VectorSubcoreMesh(core_axis_name="c", subcore_axis_name="s",
                                  num_cores=NC, num_subcores=NS)
    sc_degree = pl.kernel(
        _sc_degree_body,
        out_type=jax.ShapeDtypeStruct((NC, NPAD), jnp.float32),
        mesh=mesh,
        scratch_types=[
            pltpu.VMEM((C_CHUNKS, CH), jnp.int32),
            pltpu.VMEM((CH,), jnp.float32),
            pltpu.VMEM_SHARED((NPAD,), jnp.float32),
        ],
    )
    sc_scatter = pl.kernel(
        _sc_scatter_body,
        out_type=jax.ShapeDtypeStruct((NC, NPAD, D), jnp.float32),
        mesh=mesh,
        scratch_types=[
            pltpu.VMEM((PC, CH), jnp.int32),
            pltpu.VMEM((PC, CH), jnp.int32),
            pltpu.VMEM((NBUF * CH, D), jnp.float32),
            pltpu.VMEM_SHARED((NPAD, D), jnp.float32),
            pltpu.SemaphoreType.DMA((NBUF,)),
        ],
    )
    return sc_degree, sc_scatter


ROWS = 2048
GRID = NPAD // ROWS

_row = pl.BlockSpec((ROWS, D), lambda i: (i, 0))
_wmat = pl.BlockSpec((D, D), lambda i: (0, 0))
_vec = pl.BlockSpec((1, D), lambda i: (0, 0))
_deg = pl.BlockSpec((NC, ROWS, 1), lambda i: (0, i, 0))
_acc = pl.BlockSpec((NC, ROWS, D), lambda i: (0, i, 0))


def _dinv(deg_ref):
    return lax.rsqrt(deg_ref[0] + deg_ref[1] + 1.0)


def _elu(v):
    return jnp.where(v > 0, v, jnp.exp(jnp.minimum(v, 0.0)) - 1.0)


def _tc_mm_body(x_ref, w1_ref, ws_ref, bs_ref, h1_ref, xi_ref):
    xb = x_ref[...]
    h1_ref[...] = jnp.dot(xb, w1_ref[...], preferred_element_type=jnp.float32)
    xi_ref[...] = jnp.dot(xb, ws_ref[...],
                          preferred_element_type=jnp.float32) + bs_ref[...]


_tc_mm = pl.pallas_call(
    _tc_mm_body,
    grid=(GRID,),
    in_specs=[_row, _wmat, _wmat, _vec],
    out_specs=[_row, _row],
    out_shape=[jax.ShapeDtypeStruct((NPAD, D), jnp.float32),
               jax.ShapeDtypeStruct((NPAD, D), jnp.float32)],
)


def _tc_scale_body(h1_ref, deg_ref, hs_ref):
    hs_ref[...] = h1_ref[...] * _dinv(deg_ref)


_tc_scale = pl.pallas_call(
    _tc_scale_body,
    grid=(GRID,),
    in_specs=[_row, _deg],
    out_specs=_row,
    out_shape=jax.ShapeDtypeStruct((NPAD, D), jnp.float32),
)


def _combine(acc_ref, hs_ref, deg_ref, b_ref, g_ref, be_ref):
    di = _dinv(deg_ref)
    conv = di * (acc_ref[0] + acc_ref[1] + hs_ref[...]) + b_ref[...]
    gs = g_ref[...] * lax.rsqrt(jnp.float32(1.0 + BN_EPS))
    return _elu(conv * gs + be_ref[...])


def _tc_mid_body(acc_ref, hs_ref, deg_ref, b_ref, g_ref, be_ref, skip_ref,
                 w_ref, ws_ref, bs_ref, hsn_ref, sk_ref):
    h = _combine(acc_ref, hs_ref, deg_ref, b_ref, g_ref, be_ref) + skip_ref[...]
    di = _dinv(deg_ref)
    hsn_ref[...] = di * jnp.dot(h, w_ref[...],
                                preferred_element_type=jnp.float32)
    sk_ref[...] = jnp.dot(h, ws_ref[...],
                          preferred_element_type=jnp.float32) + bs_ref[...]


_tc_mid = pl.pallas_call(
    _tc_mid_body,
    grid=(GRID,),
    in_specs=[_acc, _row, _deg, _vec, _vec, _vec, _row, _wmat, _wmat, _vec],
    out_specs=[_row, _row],
    out_shape=[jax.ShapeDtypeStruct((NPAD, D), jnp.float32),
               jax.ShapeDtypeStruct((NPAD, D), jnp.float32)],
)


def _tc_mid2_body(acc_ref, hs_ref, deg_ref, b_ref, g_ref, be_ref, skip_ref,
                  w_ref, hsn_ref):
    h = _combine(acc_ref, hs_ref, deg_ref, b_ref, g_ref, be_ref) + skip_ref[...]
    di = _dinv(deg_ref)
    hsn_ref[...] = di * jnp.dot(h, w_ref[...],
                                preferred_element_type=jnp.float32)


_tc_mid2 = pl.pallas_call(
    _tc_mid2_body,
    grid=(GRID,),
    in_specs=[_acc, _row, _deg, _vec, _vec, _vec, _row, _wmat],
    out_specs=_row,
    out_shape=jax.ShapeDtypeStruct((NPAD, D), jnp.float32),
)


def _tc_post_body(acc_ref, hs_ref, deg_ref, b_ref, g_ref, be_ref, out_ref):
    out_ref[...] = _combine(acc_ref, hs_ref, deg_ref, b_ref, g_ref, be_ref)


_tc_post = pl.pallas_call(
    _tc_post_body,
    grid=(GRID,),
    in_specs=[_acc, _row, _deg, _vec, _vec, _vec],
    out_specs=_row,
    out_shape=jax.ShapeDtypeStruct((NPAD, D), jnp.float32),
)


def kernel(x, edge_index, W1, b1, W2, b2, W3, b3, g1, be1, g2, be2, g3, be3,
           Ws1, bs1, Ws2, bs2):
    src = edge_index[0].astype(jnp.int32)
    dst = edge_index[1].astype(jnp.int32)
    pad = DUMMY + jnp.arange(E_PAD - N_EDGES, dtype=jnp.int32) % (NPAD - DUMMY)
    src_t = jnp.concatenate([src, pad]).reshape(NW, C_CHUNKS, CH)
    dst_t = jnp.concatenate([dst, pad]).reshape(NW, C_CHUNKS, CH)
    x_p = jnp.pad(x, ((0, NPAD - N_NODES), (0, 0)))
    zeros1 = jnp.zeros((NPAD,), jnp.float32)
    row = lambda v: v.reshape(1, D)
    _sc_degree, _sc_scatter = _sc_kernels()

    deg = _sc_degree(dst_t, zeros1).reshape(NC, NPAD, 1)
    h1, x_init = _tc_mm(x_p, W1, Ws1, row(bs1))
    hs1 = _tc_scale(h1, deg)
    acc1 = _sc_scatter(hs1, src_t, dst_t)
    hs2, x_skip = _tc_mid(acc1, hs1, deg, row(b1), row(g1), row(be1), x_init,
                          W2, Ws2, row(bs2))
    acc2 = _sc_scatter(hs2, src_t, dst_t)
    hs3 = _tc_mid2(acc2, hs2, deg, row(b2), row(g2), row(be2), x_skip, W3)
    acc3 = _sc_scatter(hs3, src_t, dst_t)
    h3 = _tc_post(acc3, hs3, deg, row(b3), row(g3), row(be3))
    return h3[:N_NODES]


# speedup vs baseline: 1.2965x; 1.0069x over previous
import functools

import jax
import jax.numpy as jnp
from jax import lax
from jax.experimental import pallas as pl
from jax.experimental.pallas import tpu as pltpu, tpu_sc as plsc

N_NODES = 10000
N_EDGES = 320000
D = 128
BN_EPS = 1e-5

NPAD = 10240
DUMMY = 10000
NC, NS = 2, 16
NW = NC * NS
CH = 128
EPT = 10240
C_CHUNKS = EPT // CH
PH = 2
PC = C_CHUNKS // PH
E_PAD = NW * EPT
STRIPE = NPAD // NS

def _sc_degree_body(dst_hbm, zeros1_hbm, out_hbm, idx_v, ones_v, deg_sh):
    c = lax.axis_index("c")
    s = lax.axis_index("s")
    wid = c * NS + s
    pltpu.sync_copy(zeros1_hbm.at[pl.ds(s * STRIPE, STRIPE)],
                    deg_sh.at[pl.ds(s * STRIPE, STRIPE)])
    pltpu.sync_copy(dst_hbm.at[wid], idx_v)
    for k in range(CH // 16):
        ones_v[pl.ds(k * 16, 16)] = jnp.ones((16,), jnp.float32)
    plsc.subcore_barrier()

    def chunk(j, carry):
        pltpu.sync_copy(ones_v, deg_sh.at[idx_v.at[j]], add=True)
        return carry

    lax.fori_loop(0, C_CHUNKS, chunk, 0)
    plsc.subcore_barrier()
    pltpu.sync_copy(deg_sh.at[pl.ds(s * STRIPE, STRIPE)],
                    out_hbm.at[c, pl.ds(s * STRIPE, STRIPE)])


NBUF = 2


def _sc_scatter_body(hs_hbm, src_hbm, dst_hbm, out_hbm,
                     si_v, di_v, rows_v, acc_sh, gsem):
    c = lax.axis_index("c")
    s = lax.axis_index("s")
    wid = c * NS + s

    @pl.when(c == 0)
    def _():
        pltpu.sync_copy(hs_hbm.at[pl.ds(s * STRIPE, STRIPE)],
                        acc_sh.at[pl.ds(s * STRIPE, STRIPE)])

    @pl.when(c != 0)
    def _():
        zv = jnp.zeros((16,), jnp.float32)

        def zrow(i, carry):
            for k in range(D // 16):
                rows_v[i, pl.ds(k * 16, 16)] = zv
            return carry

        lax.fori_loop(0, NBUF * CH, zrow, 0)
        ring_rows = NBUF * CH
        off = 0
        while off < STRIPE:
            n = min(ring_rows, STRIPE - off)
            pltpu.sync_copy(rows_v.at[pl.ds(0, n)],
                            acc_sh.at[pl.ds(s * STRIPE + off, n)])
            off += n

    plsc.subcore_barrier()

    for p in range(PH):
        pltpu.sync_copy(src_hbm.at[wid, pl.ds(p * PC, PC)], si_v)
        pltpu.sync_copy(dst_hbm.at[wid, pl.ds(p * PC, PC)], di_v)
        for b in range(NBUF):
            pltpu.async_copy(hs_hbm.at[si_v.at[b]],
                             rows_v.at[pl.ds(b * CH, CH)], gsem.at[b])

        def group(g, carry):
            j0 = g * NBUF
            for b in range(NBUF):
                j = j0 + b
                pltpu.make_async_copy(hs_hbm.at[si_v.at[j]],
                                      rows_v.at[pl.ds(b * CH, CH)],
                                      gsem.at[b]).wait()
                pltpu.sync_copy(rows_v.at[pl.ds(b * CH, CH)],
                                acc_sh.at[di_v.at[j]], add=True)
                jn = j + NBUF

                @pl.when(jn < PC)
                def _():
                    pltpu.async_copy(hs_hbm.at[si_v.at[jn]],
                                     rows_v.at[pl.ds(b * CH, CH)],
                                     gsem.at[b])
            return carry

        lax.fori_loop(0, PC // NBUF, group, 0)
    plsc.subcore_barrier()
    pltpu.sync_copy(acc_sh.at[pl.ds(s * STRIPE, STRIPE)],
                    out_hbm.at[c, pl.ds(s * STRIPE, STRIPE)])


@functools.lru_cache(maxsize=1)
def _sc_kernels():
    mesh = plsc.VectorSubcoreMesh(core_axis_name="c", subcore_axis_name="s",
                                  num_cores=NC, num_subcores=NS)
    sc_degree = pl.kernel(
        _sc_degree_body,
        out_type=jax.ShapeDtypeStruct((NC, NPAD), jnp.float32),
        mesh=mesh,
        scratch_types=[
            pltpu.VMEM((C_CHUNKS, CH), jnp.int32),
            pltpu.VMEM((CH,), jnp.float32),
            pltpu.VMEM_SHARED((NPAD,), jnp.float32),
        ],
    )
    sc_scatter = pl.kernel(
        _sc_scatter_body,
        out_type=jax.ShapeDtypeStruct((NC, NPAD, D), jnp.float32),
        mesh=mesh,
        scratch_types=[
            pltpu.VMEM((PC, CH), jnp.int32),
            pltpu.VMEM((PC, CH), jnp.int32),
            pltpu.VMEM((NBUF * CH, D), jnp.float32),
            pltpu.VMEM_SHARED((NPAD, D), jnp.float32),
            pltpu.SemaphoreType.DMA((NBUF,)),
        ],
    )
    return sc_degree, sc_scatter


ROWS = 2048
GRID = NPAD // ROWS

_row = pl.BlockSpec((ROWS, D), lambda i: (i, 0))
_wmat = pl.BlockSpec((D, D), lambda i: (0, 0))
_vec = pl.BlockSpec((1, D), lambda i: (0, 0))
_deg = pl.BlockSpec((NC, ROWS, 1), lambda i: (0, i, 0))
_acc = pl.BlockSpec((NC, ROWS, D), lambda i: (0, i, 0))


def _dinv(deg_ref):
    return lax.rsqrt(deg_ref[0] + deg_ref[1] + 1.0)


def _elu(v):
    return jnp.where(v > 0, v, jnp.exp(jnp.minimum(v, 0.0)) - 1.0)


def _tc_mm_body(x_ref, w1_ref, ws_ref, bs_ref, h1_ref, xi_ref):
    xb = x_ref[...]
    h1_ref[...] = jnp.dot(xb, w1_ref[...], preferred_element_type=jnp.float32)
    xi_ref[...] = jnp.dot(xb, ws_ref[...],
                          preferred_element_type=jnp.float32) + bs_ref[...]


_tc_mm = pl.pallas_call(
    _tc_mm_body,
    grid=(GRID,),
    in_specs=[_row, _wmat, _wmat, _vec],
    out_specs=[_row, _row],
    out_shape=[jax.ShapeDtypeStruct((NPAD, D), jnp.float32),
               jax.ShapeDtypeStruct((NPAD, D), jnp.float32)],
)


def _tc_scale_body(h1_ref, deg_ref, hs_ref):
    hs_ref[...] = h1_ref[...] * _dinv(deg_ref)


_tc_scale = pl.pallas_call(
    _tc_scale_body,
    grid=(GRID,),
    in_specs=[_row, _deg],
    out_specs=_row,
    out_shape=jax.ShapeDtypeStruct((NPAD, D), jnp.float32),
)


def _combine(acc_ref, deg_ref, b_ref, g_ref, be_ref):
    di = _dinv(deg_ref)
    conv = di * (acc_ref[0] + acc_ref[1]) + b_ref[...]
    gs = g_ref[...] * lax.rsqrt(jnp.float32(1.0 + BN_EPS))
    return _elu(conv * gs + be_ref[...])


def _tc_mid_body(acc_ref, deg_ref, b_ref, g_ref, be_ref, skip_ref,
                 w_ref, ws_ref, bs_ref, hsn_ref, sk_ref):
    h = _combine(acc_ref, deg_ref, b_ref, g_ref, be_ref) + skip_ref[...]
    di = _dinv(deg_ref)
    hsn_ref[...] = di * jnp.dot(h, w_ref[...],
                                preferred_element_type=jnp.float32)
    sk_ref[...] = jnp.dot(h, ws_ref[...],
                          preferred_element_type=jnp.float32) + bs_ref[...]


_tc_mid = pl.pallas_call(
    _tc_mid_body,
    grid=(GRID,),
    in_specs=[_acc, _deg, _vec, _vec, _vec, _row, _wmat, _wmat, _vec],
    out_specs=[_row, _row],
    out_shape=[jax.ShapeDtypeStruct((NPAD, D), jnp.float32),
               jax.ShapeDtypeStruct((NPAD, D), jnp.float32)],
)


def _tc_mid2_body(acc_ref, deg_ref, b_ref, g_ref, be_ref, skip_ref,
                  w_ref, hsn_ref):
    h = _combine(acc_ref, deg_ref, b_ref, g_ref, be_ref) + skip_ref[...]
    di = _dinv(deg_ref)
    hsn_ref[...] = di * jnp.dot(h, w_ref[...],
                                preferred_element_type=jnp.float32)


_tc_mid2 = pl.pallas_call(
    _tc_mid2_body,
    grid=(GRID,),
    in_specs=[_acc, _deg, _vec, _vec, _vec, _row, _wmat],
    out_specs=_row,
    out_shape=jax.ShapeDtypeStruct((NPAD, D), jnp.float32),
)


def _tc_post_body(acc_ref, deg_ref, b_ref, g_ref, be_ref, out_ref):
    out_ref[...] = _combine(acc_ref, deg_ref, b_ref, g_ref, be_ref)


_tc_post = pl.pallas_call(
    _tc_post_body,
    grid=(GRID,),
    in_specs=[_acc, _deg, _vec, _vec, _vec],
    out_specs=_row,
    out_shape=jax.ShapeDtypeStruct((N_NODES, D), jnp.float32),
)


def kernel(x, edge_index, W1, b1, W2, b2, W3, b3, g1, be1, g2, be2, g3, be3,
           Ws1, bs1, Ws2, bs2):
    src = edge_index[0].astype(jnp.int32)
    dst = edge_index[1].astype(jnp.int32)
    pad = DUMMY + jnp.arange(E_PAD - N_EDGES, dtype=jnp.int32) % (NPAD - DUMMY)
    src_t = jnp.concatenate([src, pad]).reshape(NW, C_CHUNKS, CH)
    dst_t = jnp.concatenate([dst, pad]).reshape(NW, C_CHUNKS, CH)
    zeros1 = jnp.zeros((NPAD,), jnp.float32)
    row = lambda v: v.reshape(1, D)
    _sc_degree, _sc_scatter = _sc_kernels()

    deg = _sc_degree(dst_t, zeros1).reshape(NC, NPAD, 1)
    h1, x_init = _tc_mm(x, W1, Ws1, row(bs1))
    hs1 = _tc_scale(h1, deg)
    acc1 = _sc_scatter(hs1, src_t, dst_t)
    hs2, x_skip = _tc_mid(acc1, deg, row(b1), row(g1), row(be1), x_init,
                          W2, Ws2, row(bs2))
    acc2 = _sc_scatter(hs2, src_t, dst_t)
    hs3 = _tc_mid2(acc2, deg, row(b2), row(g2), row(be2), x_skip, W3)
    acc3 = _sc_scatter(hs3, src_t, dst_t)
    return _tc_post(acc3, deg, row(b3), row(g3), row(be3))


# speedup vs baseline: 1.3371x; 1.0313x over previous
import functools

import jax
import jax.numpy as jnp
from jax import lax
from jax.experimental import pallas as pl
from jax.experimental.pallas import tpu as pltpu, tpu_sc as plsc

N_NODES = 10000
N_EDGES = 320000
D = 128
BN_EPS = 1e-5

NPAD = 10240
DUMMY = 10000
NC, NS = 2, 16
NW = NC * NS
CH = 128
EPT = 10240
C_CHUNKS = EPT // CH
PH = 2
PC = C_CHUNKS // PH
E_PAD = NW * EPT
STRIPE = NPAD // NS

def _sc_degree_body(dst_hbm, zeros1_hbm, out_hbm, idx_v, ones_v, deg_sh):
    c = lax.axis_index("c")
    s = lax.axis_index("s")
    wid = c * NS + s
    pltpu.sync_copy(zeros1_hbm.at[pl.ds(s * STRIPE, STRIPE)],
                    deg_sh.at[pl.ds(s * STRIPE, STRIPE)])
    pltpu.sync_copy(dst_hbm.at[wid], idx_v)
    for k in range(CH // 16):
        ones_v[pl.ds(k * 16, 16)] = jnp.ones((16,), jnp.float32)
    plsc.subcore_barrier()

    def chunk(j, carry):
        pltpu.sync_copy(ones_v, deg_sh.at[idx_v.at[j]], add=True)
        return carry

    lax.fori_loop(0, C_CHUNKS, chunk, 0)
    plsc.subcore_barrier()
    pltpu.sync_copy(deg_sh.at[pl.ds(s * STRIPE, STRIPE)],
                    out_hbm.at[c, pl.ds(s * STRIPE, STRIPE)])


NBUF = 2


def _sc_scatter_body(hs_hbm, src_hbm, dst_hbm, out_hbm,
                     si_v, di_v, rows_v, acc_sh, gsem):
    c = lax.axis_index("c")
    s = lax.axis_index("s")
    wid = c * NS + s

    @pl.when(c == 0)
    def _():
        pltpu.sync_copy(hs_hbm.at[pl.ds(s * STRIPE, STRIPE)],
                        acc_sh.at[pl.ds(s * STRIPE, STRIPE)])

    @pl.when(c != 0)
    def _():
        zv = jnp.zeros((16,), jnp.float32)

        def zrow(i, carry):
            for k in range(D // 16):
                rows_v[i, pl.ds(k * 16, 16)] = zv
            return carry

        lax.fori_loop(0, NBUF * CH, zrow, 0)
        ring_rows = NBUF * CH
        off = 0
        while off < STRIPE:
            n = min(ring_rows, STRIPE - off)
            pltpu.sync_copy(rows_v.at[pl.ds(0, n)],
                            acc_sh.at[pl.ds(s * STRIPE + off, n)])
            off += n

    plsc.subcore_barrier()

    for p in range(PH):
        pltpu.sync_copy(src_hbm.at[wid, pl.ds(p * PC, PC)], si_v)
        pltpu.sync_copy(dst_hbm.at[wid, pl.ds(p * PC, PC)], di_v)
        for b in range(NBUF):
            pltpu.async_copy(hs_hbm.at[si_v.at[b]],
                             rows_v.at[pl.ds(b * CH, CH)], gsem.at[b])

        def group(g, carry):
            j0 = g * NBUF
            for b in range(NBUF):
                j = j0 + b
                pltpu.make_async_copy(hs_hbm.at[si_v.at[j]],
                                      rows_v.at[pl.ds(b * CH, CH)],
                                      gsem.at[b]).wait()
                pltpu.sync_copy(rows_v.at[pl.ds(b * CH, CH)],
                                acc_sh.at[di_v.at[j]], add=True)
                jn = j + NBUF

                @pl.when(jn < PC)
                def _():
                    pltpu.async_copy(hs_hbm.at[si_v.at[jn]],
                                     rows_v.at[pl.ds(b * CH, CH)],
                                     gsem.at[b])
            return carry

        lax.fori_loop(0, PC // NBUF, group, 0)
    plsc.subcore_barrier()
    pltpu.sync_copy(acc_sh.at[pl.ds(s * STRIPE, STRIPE)],
                    out_hbm.at[c, pl.ds(s * STRIPE, STRIPE)])


@functools.lru_cache(maxsize=1)
def _sc_kernels():
    mesh = plsc.VectorSubcoreMesh(core_axis_name="c", subcore_axis_name="s",
                                  num_cores=NC, num_subcores=NS)
    sc_degree = pl.kernel(
        _sc_degree_body,
        out_type=jax.ShapeDtypeStruct((NC, NPAD), jnp.float32),
        mesh=mesh,
        scratch_types=[
            pltpu.VMEM((C_CHUNKS, CH), jnp.int32),
            pltpu.VMEM((CH,), jnp.float32),
            pltpu.VMEM_SHARED((NPAD,), jnp.float32),
        ],
    )
    sc_scatter = pl.kernel(
        _sc_scatter_body,
        out_type=jax.ShapeDtypeStruct((NC, NPAD, D), jnp.float32),
        mesh=mesh,
        scratch_types=[
            pltpu.VMEM((PC, CH), jnp.int32),
            pltpu.VMEM((PC, CH), jnp.int32),
            pltpu.VMEM((NBUF * CH, D), jnp.float32),
            pltpu.VMEM_SHARED((NPAD, D), jnp.float32),
            pltpu.SemaphoreType.DMA((NBUF,)),
        ],
    )
    return sc_degree, sc_scatter


ROWS = 2048
GRID = NPAD // ROWS

_row = pl.BlockSpec((ROWS, D), lambda i: (i, 0))
_wmat = pl.BlockSpec((D, D), lambda i: (0, 0))
_vec = pl.BlockSpec((1, D), lambda i: (0, 0))
_acc = pl.BlockSpec((NC, ROWS, D), lambda i: (0, i, 0))


def _elu(v):
    return jnp.where(v > 0, v, jnp.exp(jnp.minimum(v, 0.0)) - 1.0)


def _tc_mm_body(x_ref, w1_ref, ws_ref, bs_ref, h1_ref, xi_ref):
    xb = x_ref[...]
    h1_ref[...] = jnp.dot(xb, w1_ref[...], preferred_element_type=jnp.float32)
    xi_ref[...] = jnp.dot(xb, ws_ref[...],
                          preferred_element_type=jnp.float32) + bs_ref[...]


_tc_mm = pl.pallas_call(
    _tc_mm_body,
    grid=(GRID,),
    in_specs=[_row, _wmat, _wmat, _vec],
    out_specs=[_row, _row],
    out_shape=[jax.ShapeDtypeStruct((NPAD, D), jnp.float32),
               jax.ShapeDtypeStruct((NPAD, D), jnp.float32)],
)


def _tc_scale_body(h1_ref, deg_ref, hs_ref, dinvb_ref):
    degsum = deg_ref[0] + deg_ref[1] + 1.0
    di = lax.rsqrt(jnp.reshape(degsum, (NPAD, 1)))
    dinv_b = jnp.broadcast_to(di, (NPAD, D))
    dinvb_ref[...] = dinv_b
    hs_ref[...] = h1_ref[...] * dinv_b


_tc_scale = pl.pallas_call(
    _tc_scale_body,
    grid=(1,),
    in_specs=[pl.BlockSpec((NPAD, D), lambda i: (0, 0)),
              pl.BlockSpec((NC, NPAD), lambda i: (0, 0))],
    out_specs=[pl.BlockSpec((NPAD, D), lambda i: (0, 0)),
               pl.BlockSpec((NPAD, D), lambda i: (0, 0))],
    out_shape=[jax.ShapeDtypeStruct((NPAD, D), jnp.float32),
               jax.ShapeDtypeStruct((NPAD, D), jnp.float32)],
)


def _combine(acc_ref, dinv_ref, b_ref, g_ref, be_ref):
    conv = dinv_ref[...] * (acc_ref[0] + acc_ref[1]) + b_ref[...]
    gs = g_ref[...] * lax.rsqrt(jnp.float32(1.0 + BN_EPS))
    return _elu(conv * gs + be_ref[...])


def _tc_mid_body(acc_ref, dinv_ref, b_ref, g_ref, be_ref, skip_ref,
                 w_ref, ws_ref, bs_ref, hsn_ref, sk_ref):
    h = _combine(acc_ref, dinv_ref, b_ref, g_ref, be_ref) + skip_ref[...]
    hsn_ref[...] = dinv_ref[...] * jnp.dot(h, w_ref[...],
                                           preferred_element_type=jnp.float32)
    sk_ref[...] = jnp.dot(h, ws_ref[...],
                          preferred_element_type=jnp.float32) + bs_ref[...]


_tc_mid = pl.pallas_call(
    _tc_mid_body,
    grid=(GRID,),
    in_specs=[_acc, _row, _vec, _vec, _vec, _row, _wmat, _wmat, _vec],
    out_specs=[_row, _row],
    out_shape=[jax.ShapeDtypeStruct((NPAD, D), jnp.float32),
               jax.ShapeDtypeStruct((NPAD, D), jnp.float32)],
)


def _tc_mid2_body(acc_ref, dinv_ref, b_ref, g_ref, be_ref, skip_ref,
                  w_ref, hsn_ref):
    h = _combine(acc_ref, dinv_ref, b_ref, g_ref, be_ref) + skip_ref[...]
    hsn_ref[...] = dinv_ref[...] * jnp.dot(h, w_ref[...],
                                           preferred_element_type=jnp.float32)


_tc_mid2 = pl.pallas_call(
    _tc_mid2_body,
    grid=(GRID,),
    in_specs=[_acc, _row, _vec, _vec, _vec, _row, _wmat],
    out_specs=_row,
    out_shape=jax.ShapeDtypeStruct((NPAD, D), jnp.float32),
)


def _tc_post_body(acc_ref, dinv_ref, b_ref, g_ref, be_ref, out_ref):
    out_ref[...] = _combine(acc_ref, dinv_ref, b_ref, g_ref, be_ref)


_tc_post = pl.pallas_call(
    _tc_post_body,
    grid=(GRID,),
    in_specs=[_acc, _row, _vec, _vec, _vec],
    out_specs=_row,
    out_shape=jax.ShapeDtypeStruct((N_NODES, D), jnp.float32),
)


def kernel(x, edge_index, W1, b1, W2, b2, W3, b3, g1, be1, g2, be2, g3, be3,
           Ws1, bs1, Ws2, bs2):
    src = edge_index[0].astype(jnp.int32)
    dst = edge_index[1].astype(jnp.int32)
    pad = DUMMY + jnp.arange(E_PAD - N_EDGES, dtype=jnp.int32) % (NPAD - DUMMY)
    src_t = jnp.concatenate([src, pad]).reshape(NW, C_CHUNKS, CH)
    dst_t = jnp.concatenate([dst, pad]).reshape(NW, C_CHUNKS, CH)
    zeros1 = jnp.zeros((NPAD,), jnp.float32)
    row = lambda v: v.reshape(1, D)
    _sc_degree, _sc_scatter = _sc_kernels()

    deg = _sc_degree(dst_t, zeros1)
    h1, x_init = _tc_mm(x, W1, Ws1, row(bs1))
    hs1, dinv_b = _tc_scale(h1, deg)
    acc1 = _sc_scatter(hs1, src_t, dst_t)
    hs2, x_skip = _tc_mid(acc1, dinv_b, row(b1), row(g1), row(be1), x_init,
                          W2, Ws2, row(bs2))
    acc2 = _sc_scatter(hs2, src_t, dst_t)
    hs3 = _tc_mid2(acc2, dinv_b, row(b2), row(g2), row(be2), x_skip, W3)
    acc3 = _sc_scatter(hs3, src_t, dst_t)
    return _tc_post(acc3, dinv_b, row(b3), row(g3), row(be3))


# speedup vs baseline: 1.3493x; 1.0091x over previous
import functools

import jax
import jax.numpy as jnp
import numpy as np
from jax import lax
from jax.experimental import pallas as pl
from jax.experimental.pallas import tpu as pltpu, tpu_sc as plsc

N_NODES = 10000
N_EDGES = 320000
D = 128
BN_EPS = 1e-5

NPAD = 10240
DUMMY = 10000
NC, NS = 2, 16
NW = NC * NS
CH = 128
EPT = 10240
C_CHUNKS = EPT // CH
PH = 2
PC = C_CHUNKS // PH
E_PAD = NW * EPT
STRIPE = NPAD // NS

_PAD2 = np.broadcast_to(
    np.asarray(DUMMY + np.arange(E_PAD - N_EDGES) % (NPAD - DUMMY),
               np.int32), (2, E_PAD - N_EDGES)).copy()

def _sc_degree_body(dst_hbm, zeros1_hbm, ones_hbm, out_hbm, idx_v, ones_v,
                    deg_sh):
    c = lax.axis_index("c")
    s = lax.axis_index("s")
    wid = c * NS + s
    pltpu.sync_copy(zeros1_hbm.at[pl.ds(s * STRIPE, STRIPE)],
                    deg_sh.at[pl.ds(s * STRIPE, STRIPE)])
    pltpu.sync_copy(dst_hbm.at[wid], idx_v)
    pltpu.sync_copy(ones_hbm, ones_v)
    plsc.subcore_barrier()

    def chunk(j, carry):
        pltpu.sync_copy(ones_v.at[j], deg_sh.at[idx_v.at[j]], add=True)
        return carry

    lax.fori_loop(0, C_CHUNKS, chunk, 0)
    plsc.subcore_barrier()
    pltpu.sync_copy(deg_sh.at[pl.ds(s * STRIPE, STRIPE)],
                    out_hbm.at[c, pl.ds(s * STRIPE, STRIPE)])


NBUF = 2


def _sc_scatter_body(hs_hbm, src_hbm, dst_hbm, out_hbm,
                     si_v, di_v, rows_v, acc_sh, gsem):
    c = lax.axis_index("c")
    s = lax.axis_index("s")
    wid = c * NS + s

    @pl.when(c == 0)
    def _():
        pltpu.sync_copy(hs_hbm.at[pl.ds(s * STRIPE, STRIPE)],
                        acc_sh.at[pl.ds(s * STRIPE, STRIPE)])

    @pl.when(c != 0)
    def _():
        zv = jnp.zeros((16,), jnp.float32)

        def zrow(i, carry):
            for k in range(D // 16):
                rows_v[i, pl.ds(k * 16, 16)] = zv
            return carry

        lax.fori_loop(0, NBUF * CH, zrow, 0)
        ring_rows = NBUF * CH
        off = 0
        while off < STRIPE:
            n = min(ring_rows, STRIPE - off)
            pltpu.sync_copy(rows_v.at[pl.ds(0, n)],
                            acc_sh.at[pl.ds(s * STRIPE + off, n)])
            off += n

    plsc.subcore_barrier()

    for p in range(PH):
        pltpu.sync_copy(src_hbm.at[wid, pl.ds(p * PC, PC)], si_v)
        pltpu.sync_copy(dst_hbm.at[wid, pl.ds(p * PC, PC)], di_v)
        for b in range(NBUF):
            pltpu.async_copy(hs_hbm.at[si_v.at[b]],
                             rows_v.at[pl.ds(b * CH, CH)], gsem.at[b])

        def group(g, carry):
            j0 = g * NBUF
            for b in range(NBUF):
                j = j0 + b
                pltpu.make_async_copy(hs_hbm.at[si_v.at[j]],
                                      rows_v.at[pl.ds(b * CH, CH)],
                                      gsem.at[b]).wait()
                pltpu.sync_copy(rows_v.at[pl.ds(b * CH, CH)],
                                acc_sh.at[di_v.at[j]], add=True)
                jn = j + NBUF

                @pl.when(jn < PC)
                def _():
                    pltpu.async_copy(hs_hbm.at[si_v.at[jn]],
                                     rows_v.at[pl.ds(b * CH, CH)],
                                     gsem.at[b])
            return carry

        lax.fori_loop(0, PC // NBUF, group, 0)
    plsc.subcore_barrier()
    pltpu.sync_copy(acc_sh.at[pl.ds(s * STRIPE, STRIPE)],
                    out_hbm.at[c, pl.ds(s * STRIPE, STRIPE)])


@functools.lru_cache(maxsize=1)
def _sc_kernels():
    mesh = plsc.VectorSubcoreMesh(core_axis_name="c", subcore_axis_name="s",
                                  num_cores=NC, num_subcores=NS)
    sc_degree = pl.kernel(
        _sc_degree_body,
        out_type=jax.ShapeDtypeStruct((NC, NPAD), jnp.float32),
        mesh=mesh,
        scratch_types=[
            pltpu.VMEM((C_CHUNKS, CH), jnp.int32),
            pltpu.VMEM((C_CHUNKS, CH), jnp.float32),
            pltpu.VMEM_SHARED((NPAD,), jnp.float32),
        ],
    )
    sc_scatter = pl.kernel(
        _sc_scatter_body,
        out_type=jax.ShapeDtypeStruct((NC, NPAD, D), jnp.float32),
        mesh=mesh,
        scratch_types=[
            pltpu.VMEM((PC, CH), jnp.int32),
            pltpu.VMEM((PC, CH), jnp.int32),
            pltpu.VMEM((NBUF * CH, D), jnp.float32),
            pltpu.VMEM_SHARED((NPAD, D), jnp.float32),
            pltpu.SemaphoreType.DMA((NBUF,)),
        ],
    )
    return sc_degree, sc_scatter


ROWS = 2048
GRID = NPAD // ROWS

_row = pl.BlockSpec((ROWS, D), lambda i: (i, 0))
_wmat = pl.BlockSpec((D, D), lambda i: (0, 0))
_vec = pl.BlockSpec((1, D), lambda i: (0, 0))
_acc = pl.BlockSpec((NC, ROWS, D), lambda i: (0, i, 0))


def _elu(v):
    return jnp.where(v > 0, v, jnp.exp(jnp.minimum(v, 0.0)) - 1.0)


def _tc_mm_body(x_ref, w1_ref, ws_ref, bs_ref, h1_ref, xi_ref):
    xb = x_ref[...]
    h1_ref[...] = jnp.dot(xb, w1_ref[...], preferred_element_type=jnp.float32)
    xi_ref[...] = jnp.dot(xb, ws_ref[...],
                          preferred_element_type=jnp.float32) + bs_ref[...]


_tc_mm = pl.pallas_call(
    _tc_mm_body,
    grid=(GRID,),
    in_specs=[_row, _wmat, _wmat, _vec],
    out_specs=[_row, _row],
    out_shape=[jax.ShapeDtypeStruct((NPAD, D), jnp.float32),
               jax.ShapeDtypeStruct((NPAD, D), jnp.float32)],
)


def _tc_scale_body(h1_ref, deg_ref, hs_ref, dinvb_ref):
    degsum = deg_ref[0] + deg_ref[1] + 1.0
    di = lax.rsqrt(jnp.reshape(degsum, (NPAD, 1)))
    dinv_b = jnp.broadcast_to(di, (NPAD, D))
    dinvb_ref[...] = dinv_b
    hs_ref[...] = h1_ref[...] * dinv_b


_tc_scale = pl.pallas_call(
    _tc_scale_body,
    grid=(1,),
    in_specs=[pl.BlockSpec((NPAD, D), lambda i: (0, 0)),
              pl.BlockSpec((NC, NPAD), lambda i: (0, 0))],
    out_specs=[pl.BlockSpec((NPAD, D), lambda i: (0, 0)),
               pl.BlockSpec((NPAD, D), lambda i: (0, 0))],
    out_shape=[jax.ShapeDtypeStruct((NPAD, D), jnp.float32),
               jax.ShapeDtypeStruct((NPAD, D), jnp.float32)],
)


def _combine(acc_ref, dinv_ref, b_ref, g_ref, be_ref):
    conv = dinv_ref[...] * (acc_ref[0] + acc_ref[1]) + b_ref[...]
    gs = g_ref[...] * lax.rsqrt(jnp.float32(1.0 + BN_EPS))
    return _elu(conv * gs + be_ref[...])


def _tc_mid_body(acc_ref, dinv_ref, b_ref, g_ref, be_ref, skip_ref,
                 w_ref, ws_ref, bs_ref, hsn_ref, sk_ref):
    h = _combine(acc_ref, dinv_ref, b_ref, g_ref, be_ref) + skip_ref[...]
    hsn_ref[...] = dinv_ref[...] * jnp.dot(h, w_ref[...],
                                           preferred_element_type=jnp.float32)
    sk_ref[...] = jnp.dot(h, ws_ref[...],
                          preferred_element_type=jnp.float32) + bs_ref[...]


_tc_mid = pl.pallas_call(
    _tc_mid_body,
    grid=(GRID,),
    in_specs=[_acc, _row, _vec, _vec, _vec, _row, _wmat, _wmat, _vec],
    out_specs=[_row, _row],
    out_shape=[jax.ShapeDtypeStruct((NPAD, D), jnp.float32),
               jax.ShapeDtypeStruct((NPAD, D), jnp.float32)],
)


def _tc_mid2_body(acc_ref, dinv_ref, b_ref, g_ref, be_ref, skip_ref,
                  w_ref, hsn_ref):
    h = _combine(acc_ref, dinv_ref, b_ref, g_ref, be_ref) + skip_ref[...]
    hsn_ref[...] = dinv_ref[...] * jnp.dot(h, w_ref[...],
                                           preferred_element_type=jnp.float32)


_tc_mid2 = pl.pallas_call(
    _tc_mid2_body,
    grid=(GRID,),
    in_specs=[_acc, _row, _vec, _vec, _vec, _row, _wmat],
    out_specs=_row,
    out_shape=jax.ShapeDtypeStruct((NPAD, D), jnp.float32),
)


def _tc_post_body(acc_ref, dinv_ref, b_ref, g_ref, be_ref, out_ref):
    out_ref[...] = _combine(acc_ref, dinv_ref, b_ref, g_ref, be_ref)


_tc_post = pl.pallas_call(
    _tc_post_body,
    grid=(GRID,),
    in_specs=[_acc, _row, _vec, _vec, _vec],
    out_specs=_row,
    out_shape=jax.ShapeDtypeStruct((N_NODES, D), jnp.float32),
)


def kernel(x, edge_index, W1, b1, W2, b2, W3, b3, g1, be1, g2, be2, g3, be3,
           Ws1, bs1, Ws2, bs2):
    ei_p = jnp.concatenate([edge_index.astype(jnp.int32), _PAD2], axis=1)
    src_t = ei_p[0].reshape(NW, C_CHUNKS, CH)
    dst_t = ei_p[1].reshape(NW, C_CHUNKS, CH)
    zeros1 = jnp.zeros((NPAD,), jnp.float32)
    ones_t = jnp.ones((C_CHUNKS, CH), jnp.float32)
    row = lambda v: v.reshape(1, D)
    _sc_degree, _sc_scatter = _sc_kernels()

    deg = _sc_degree(dst_t, zeros1, ones_t)
    h1, x_init = _tc_mm(x, W1, Ws1, row(bs1))
    hs1, dinv_b = _tc_scale(h1, deg)
    acc1 = _sc_scatter(hs1, src_t, dst_t)
    hs2, x_skip = _tc_mid(acc1, dinv_b, row(b1), row(g1), row(be1), x_init,
                          W2, Ws2, row(bs2))
    acc2 = _sc_scatter(hs2, src_t, dst_t)
    hs3 = _tc_mid2(acc2, dinv_b, row(b2), row(g2), row(be2), x_skip, W3)
    acc3 = _sc_scatter(hs3, src_t, dst_t)
    return _tc_post(acc3, dinv_b, row(b3), row(g3), row(be3))
